# Initial kernel scaffold; baseline (speedup 1.0000x reference)
import jax
import jax.numpy as jnp
from jax.experimental import pallas as pl


def kernel(x, edge_index_all, gamma1, beta1, W1, b1, gamma2, beta2, W2, b2, Wout, bout):
    raise NotImplementedError("write your pallas kernel here")


# speedup vs baseline: 4.8601x; 4.8601x over previous
import functools

import jax
import jax.numpy as jnp
from jax import lax
from jax.experimental import pallas as pl
from jax.experimental.pallas import tpu as pltpu
from jax.experimental.pallas import tpu_sc as plsc

N_NODES = 10000
H_DIM = 128
CHUNK = 128
NC = 2
NS = 16
NW = NC * NS
DEGW = 128
N_PAD = 10240
EPS = 1e-5


@functools.partial(jax.jit, static_argnames=("n_chunks",))
def _sc_aggregate(h, src_idx, dst_idx, n_chunks):
    n = N_PAD
    rows_per_sub = n // NS
    mesh = plsc.VectorSubcoreMesh(core_axis_name="c", subcore_axis_name="s")

    scratch = [
        pltpu.VMEM((n_chunks, CHUNK), jnp.int32),
        pltpu.VMEM((n_chunks, CHUNK), jnp.int32),
        pltpu.VMEM((CHUNK, H_DIM), jnp.float32),
        pltpu.VMEM_SHARED((n, H_DIM), jnp.float32),
        pltpu.SemaphoreType.DMA,
    ]

    def body(h_hbm, src_hbm, dst_hbm, zrows_hbm, agg_out,
             src_v, dst_v, rowbuf, agg_sh, sem):
        cid = lax.axis_index("c")
        sid = lax.axis_index("s")
        wid = cid * NS + sid

        pltpu.sync_copy(src_hbm.at[wid], src_v)
        pltpu.sync_copy(dst_hbm.at[wid], dst_v)
        base = sid * rows_per_sub
        pltpu.sync_copy(zrows_hbm, agg_sh.at[pl.ds(base, rows_per_sub)])
        plsc.subcore_barrier()

        def step(j, carry):
            pltpu.async_copy(h_hbm.at[src_v.at[j]], rowbuf, sem).wait()
            pltpu.sync_copy(rowbuf, agg_sh.at[dst_v.at[j]], add=True)
            return carry

        lax.fori_loop(0, n_chunks, step, 0)
        plsc.subcore_barrier()

        out_base = cid * n + base
        pltpu.sync_copy(agg_sh.at[pl.ds(base, rows_per_sub)],
                        agg_out.at[pl.ds(out_base, rows_per_sub)])

    zrows = jnp.zeros((rows_per_sub, H_DIM), jnp.float32)
    fn = pl.kernel(body, out_type=jax.ShapeDtypeStruct((NC * n, H_DIM),
                                                       jnp.float32),
                   mesh=mesh, scratch_types=scratch)
    return fn(h, src_idx, dst_idx, zrows)


@functools.partial(jax.jit, static_argnames=("n_chunks",))
def _sc_degree(dst_idx, n_chunks):
    n = N_PAD
    rows_per_sub = n // NS
    mesh = plsc.VectorSubcoreMesh(core_axis_name="c", subcore_axis_name="s")

    scratch = [
        pltpu.VMEM((n_chunks, CHUNK), jnp.int32),
        pltpu.VMEM((CHUNK, DEGW), jnp.float32),
        pltpu.VMEM_SHARED((n, DEGW), jnp.float32),
    ]

    def body(dst_hbm, zdeg_hbm, ones_hbm, deg_out, dst_v, ones_v, deg_sh):
        cid = lax.axis_index("c")
        sid = lax.axis_index("s")
        wid = cid * NS + sid

        pltpu.sync_copy(dst_hbm.at[wid], dst_v)
        pltpu.sync_copy(ones_hbm, ones_v)
        base = sid * rows_per_sub
        pltpu.sync_copy(zdeg_hbm, deg_sh.at[pl.ds(base, rows_per_sub)])
        plsc.subcore_barrier()

        def step(j, carry):
            pltpu.sync_copy(ones_v, deg_sh.at[dst_v.at[j]], add=True)
            return carry

        lax.fori_loop(0, n_chunks, step, 0)
        plsc.subcore_barrier()

        out_base = cid * n + base
        pltpu.sync_copy(deg_sh.at[pl.ds(base, rows_per_sub)],
                        deg_out.at[pl.ds(out_base, rows_per_sub)])

    zdeg = jnp.zeros((rows_per_sub, DEGW), jnp.float32)
    ones = jnp.ones((CHUNK, DEGW), jnp.float32)
    fn = pl.kernel(body, out_type=jax.ShapeDtypeStruct((NC * n, DEGW),
                                                       jnp.float32),
                   mesh=mesh, scratch_types=scratch)
    return fn(dst_idx, zdeg, ones)


def _bn(x, gamma, beta):
    mu = jnp.mean(x, axis=0, keepdims=True)
    xc = x - mu
    var = jnp.mean(xc * xc, axis=0, keepdims=True)
    return xc * lax.rsqrt(var + EPS) * gamma + beta


def _tc1_body(x_ref, g_ref, be_ref, w_ref, o_ref):
    xh = _bn(x_ref[...], g_ref[...], be_ref[...])
    o_ref[...] = jnp.dot(xh, w_ref[...], preferred_element_type=jnp.float32)


def _tc2_body(pa_ref, pb_ref, da_ref, db_ref, b_ref, g_ref, be_ref, w_ref,
              o_ref):
    deg = jnp.maximum(da_ref[...][:, :1] + db_ref[...][:, :1], 1.0)
    agg = (pa_ref[...] + pb_ref[...]) / deg
    o1 = jnp.maximum(agg + b_ref[...], 0.0)
    xh = _bn(o1, g_ref[...], be_ref[...])
    o_ref[...] = jnp.dot(xh, w_ref[...], preferred_element_type=jnp.float32)


def _tc3_body(pa_ref, pb_ref, da_ref, db_ref, b_ref, w_ref, bo_ref, o_ref):
    deg = jnp.maximum(da_ref[...][:, :1] + db_ref[...][:, :1], 1.0)
    agg = (pa_ref[...] + pb_ref[...]) / deg
    o2 = jnp.maximum(agg + b_ref[...], 0.0)
    logits = jnp.dot(o2, w_ref[...], preferred_element_type=jnp.float32)
    logits = logits + bo_ref[...]
    m = jnp.max(logits, axis=1, keepdims=True)
    z = logits - m
    lse = jnp.log(jnp.sum(jnp.exp(z), axis=1, keepdims=True))
    o_ref[...] = z - lse


def _dense(body, out_cols, *args):
    return pl.pallas_call(
        body,
        out_shape=jax.ShapeDtypeStruct((N_NODES, out_cols), jnp.float32),
    )(*args)


def kernel(x, edge_index_all, gamma1, beta1, W1, b1, gamma2, beta2, W2, b2,
           Wout, bout):
    n = x.shape[0]
    e = edge_index_all.shape[1]
    per_tile_chunks = -(-e // (NW * CHUNK))
    e_pad = NW * per_tile_chunks * CHUNK

    src = edge_index_all[0]
    dst = edge_index_all[1]
    if e_pad != e:
        pad = e_pad - e
        src = jnp.concatenate([src, jnp.zeros((pad,), jnp.int32)])
        dst = jnp.concatenate([dst, jnp.full((pad,), n, jnp.int32)])
    src = src.reshape(NW, per_tile_chunks, CHUNK)
    dst = dst.reshape(NW, per_tile_chunks, CHUNK)

    g1 = gamma1.reshape(1, -1)
    be1 = beta1.reshape(1, -1)
    g2 = gamma2.reshape(1, -1)
    be2 = beta2.reshape(1, -1)
    bb1 = b1.reshape(1, -1)
    bb2 = b2.reshape(1, -1)
    bo = bout.reshape(1, -1)

    h1 = _dense(_tc1_body, H_DIM, x, g1, be1, W1)
    degp = _sc_degree(dst, n_chunks=per_tile_chunks)
    p1 = _sc_aggregate(h1, src, dst, n_chunks=per_tile_chunks)
    p1a, p1b = p1[:n], p1[N_PAD:N_PAD + n]
    da, db = degp[:n], degp[N_PAD:N_PAD + n]
    h2 = _dense(_tc2_body, H_DIM, p1a, p1b, da, db, bb1, g2, be2, W2)
    p2 = _sc_aggregate(h2, src, dst, n_chunks=per_tile_chunks)
    p2a, p2b = p2[:n], p2[N_PAD:N_PAD + n]
    out = _dense(_tc3_body, Wout.shape[1], p2a, p2b, da, db, bb2, Wout, bo)
    return out
